# baseline (speedup 1.0000x reference)
import functools

import jax
import jax.numpy as jnp
from jax import lax
from jax.experimental import pallas as pl
from jax.experimental.pallas import tpu as pltpu

L = 24
M = 64
N = 1024
S = 8
B = 2048
BB = 256


def _matmul_sc(Ab, Bb2, n, m, kdim):
    out = [[None] * m for _ in range(n)]
    for i in range(n):
        for j in range(m):
            acc = Ab[i][0] * Bb2[0][j]
            for q in range(1, kdim):
                acc = acc + Ab[i][q] * Bb2[q][j]
            out[i][j] = acc
    return out


def _inv_unit_lower(Lb, n):
    if n == 1:
        return [[jnp.ones_like(Lb[0][0])]]
    h = n // 2
    iA = _inv_unit_lower([row[:h] for row in Lb[:h]], h)
    iB = _inv_unit_lower([row[h:] for row in Lb[h:]], n - h)
    Cb = [row[:h] for row in Lb[h:]]
    X = _matmul_sc(iB, _matmul_sc(Cb, iA, n - h, h, h), n - h, h, n - h)
    z = jnp.zeros_like(Lb[0][0])
    out = [[z] * n for _ in range(n)]
    for i in range(h):
        for j in range(h):
            out[i][j] = iA[i][j]
    for i in range(n - h):
        for j in range(h):
            out[h + i][j] = -X[i][j]
        for j in range(n - h):
            out[h + i][h + j] = iB[i][j]
    return out


def _inv_upper(Ub, n):
    if n == 1:
        return [[1.0 / Ub[0][0]]]
    h = n // 2
    iA = _inv_upper([row[:h] for row in Ub[:h]], h)
    iB = _inv_upper([row[h:] for row in Ub[h:]], n - h)
    Cb = [row[h:] for row in Ub[:h]]
    X = _matmul_sc(_matmul_sc(iA, Cb, h, n - h, h), iB, h, n - h, n - h)
    z = jnp.zeros_like(Ub[0][0])
    out = [[z] * n for _ in range(n)]
    for i in range(h):
        for j in range(h):
            out[i][j] = iA[i][j]
        for j in range(n - h):
            out[i][h + j] = -X[i][j]
    for i in range(n - h):
        for j in range(n - h):
            out[h + i][h + j] = iB[i][j]
    return out


def _lu_solve_vec(g, rhs, n):
    a = [[(g[i][j] if j <= i else g[j][i]) for j in range(n)]
         for i in range(n)]
    b = [rhs[i] for i in range(n)]
    for k in range(n):
        mx = jnp.abs(a[k][k])
        for i in range(k + 1, n):
            mx = jnp.maximum(mx, jnp.abs(a[i][k]))
        taken = jnp.zeros_like(mx, dtype=bool)
        newk = [jnp.zeros_like(a[k][j]) for j in range(n)]
        newbk = jnp.zeros_like(b[k])
        for i in range(k, n):
            is_p = (jnp.abs(a[i][k]) == mx) & (~taken)
            taken = taken | is_p
            for j in range(n):
                newk[j] = jnp.where(is_p, a[i][j], newk[j])
            newbk = jnp.where(is_p, b[i], newbk)
            for j in range(n):
                a[i][j] = jnp.where(is_p, a[k][j], a[i][j])
            b[i] = jnp.where(is_p, b[k], b[i])
        for j in range(n):
            a[k][j] = newk[j]
        b[k] = newbk
        inv = 1.0 / a[k][k]
        for i in range(k + 1, n):
            mfac = a[i][k] * inv
            a[i][k] = mfac
            for j in range(k + 1, n):
                a[i][j] = a[i][j] - mfac * a[k][j]
    one = jnp.ones_like(a[0][0])
    zero = jnp.zeros_like(a[0][0])
    Lmat = [[a[i][j] if j < i else (one if j == i else zero)
             for j in range(n)] for i in range(n)]
    Umat = [[a[i][j] if j >= i else zero for j in range(n)]
            for i in range(n)]
    iL = _inv_unit_lower(Lmat, n)
    zv = [_matmul_sc(iL, [[v] for v in b], n, 1, n)[i][0] for i in range(n)]
    iU = _inv_upper(Umat, n)
    return [_matmul_sc(iU, [[v] for v in zv], n, 1, n)[i][0]
            for i in range(n)]


def _omp_block(kt, Dm, Dt):
    Bb = kt.shape[0]
    f32 = jnp.float32
    bf16 = jnp.bfloat16
    iota = lax.broadcasted_iota(jnp.int32, (Bb, N), 1)
    D16 = Dm.astype(bf16)

    r = kt
    idxs = []
    dcols = []
    onehots = []
    rhs = []
    g = [[None] * S for _ in range(S)]
    alpha0 = None
    x = None
    recon = None
    for t in range(S):
        proj = jnp.dot(r.astype(bf16), D16,
                       preferred_element_type=f32)
        if t == 0:
            alpha0 = proj
        a = jnp.abs(proj)
        m = jnp.max(a, axis=1, keepdims=True)
        idx = jnp.min(jnp.where(a == m, iota, N), axis=1)
        idxs.append(idx)
        onehot = (iota == idx[:, None]).astype(f32)
        dcol = jnp.dot(onehot, Dt, preferred_element_type=f32,
                       precision=lax.Precision.HIGHEST)
        dcols.append(dcol)
        onehots.append(onehot)
        rhs.append(jnp.sum(alpha0 * onehot, axis=1))
        grow = jnp.dot(dcol.astype(bf16), D16, preferred_element_type=f32)
        for j in range(t + 1):
            gv = jnp.sum(grow * onehots[j], axis=1)
            if j == t:
                gv = gv + 1e-6
            g[t][j] = gv
        x = _lu_solve_vec(g, rhs, t + 1)
        recon = dcols[0] * x[0][:, None]
        for j in range(1, t + 1):
            recon = recon + dcols[j] * x[j][:, None]
        r = kt - recon
    return idxs, x, recon, r, iota


def _kernel_body(kt_ref, d_ref, dt_ref, y_ref, khat_ref, loss_ref):
    kt = kt_ref[0]
    Dm = d_ref[0]
    idxs, x, recon, resid, iota = _omp_block(kt, Dm, dt_ref[0])

    y = jnp.zeros((kt.shape[0], N), jnp.float32)
    for t in range(S):
        y = jnp.where(iota == idxs[t][:, None], x[t][:, None], y)
    y_ref[...] = y
    khat_ref[0] = recon

    li = pl.program_id(0)
    bi = pl.program_id(1)

    @pl.when((li == 0) & (bi == 0))
    def _():
        loss_ref[0, 0] = 0.0

    loss_ref[0, 0] += jnp.sum(resid * resid)

    @pl.when((li == L - 1) & (bi == pl.num_programs(1) - 1))
    def _():
        loss_ref[0, 0] = loss_ref[0, 0] / (B * L * M)


@functools.partial(jax.jit, static_argnames=("interpret",))
def kernel(k, D, interpret=False):
    ktr = jnp.transpose(k, (1, 0, 2))
    Dt = jnp.transpose(D, (0, 2, 1))
    nb = B // BB
    y, khat, loss = pl.pallas_call(
        _kernel_body,
        grid=(L, nb),
        in_specs=[
            pl.BlockSpec((1, BB, M), lambda l, b: (l, b, 0)),
            pl.BlockSpec((1, M, N), lambda l, b: (l, 0, 0)),
            pl.BlockSpec((1, N, M), lambda l, b: (l, 0, 0)),
        ],
        out_specs=[
            pl.BlockSpec((BB, N), lambda l, b: (b, l)),
            pl.BlockSpec((1, BB, M), lambda l, b: (l, b, 0)),
            pl.BlockSpec(memory_space=pltpu.SMEM, block_shape=(1, 1),
                         index_map=lambda l, b: (0, 0)),
        ],
        out_shape=[
            jax.ShapeDtypeStruct((B, L * N), jnp.float32),
            jax.ShapeDtypeStruct((L, B, M), jnp.float32),
            jax.ShapeDtypeStruct((1, 1), jnp.float32),
        ],
        interpret=interpret,
    )(ktr, D, Dt)
    return (loss[0, 0], jnp.transpose(khat, (1, 0, 2)),
            jnp.reshape(y, (B, L, N)))


# speedup vs baseline: 1.1104x; 1.1104x over previous
import functools

import jax
import jax.numpy as jnp
from jax import lax
from jax.experimental import pallas as pl
from jax.experimental.pallas import tpu as pltpu

L = 24
M = 64
N = 1024
S = 8
B = 2048
BB = 512


def _matmul_sc(Ab, Bb2, n, m, kdim):
    out = [[None] * m for _ in range(n)]
    for i in range(n):
        for j in range(m):
            acc = Ab[i][0] * Bb2[0][j]
            for q in range(1, kdim):
                acc = acc + Ab[i][q] * Bb2[q][j]
            out[i][j] = acc
    return out


def _inv_unit_lower(Lb, n):
    if n == 1:
        return [[jnp.ones_like(Lb[0][0])]]
    h = n // 2
    iA = _inv_unit_lower([row[:h] for row in Lb[:h]], h)
    iB = _inv_unit_lower([row[h:] for row in Lb[h:]], n - h)
    Cb = [row[:h] for row in Lb[h:]]
    X = _matmul_sc(iB, _matmul_sc(Cb, iA, n - h, h, h), n - h, h, n - h)
    z = jnp.zeros_like(Lb[0][0])
    out = [[z] * n for _ in range(n)]
    for i in range(h):
        for j in range(h):
            out[i][j] = iA[i][j]
    for i in range(n - h):
        for j in range(h):
            out[h + i][j] = -X[i][j]
        for j in range(n - h):
            out[h + i][h + j] = iB[i][j]
    return out


def _inv_upper(Ub, n):
    if n == 1:
        return [[1.0 / Ub[0][0]]]
    h = n // 2
    iA = _inv_upper([row[:h] for row in Ub[:h]], h)
    iB = _inv_upper([row[h:] for row in Ub[h:]], n - h)
    Cb = [row[h:] for row in Ub[:h]]
    X = _matmul_sc(_matmul_sc(iA, Cb, h, n - h, h), iB, h, n - h, n - h)
    z = jnp.zeros_like(Ub[0][0])
    out = [[z] * n for _ in range(n)]
    for i in range(h):
        for j in range(h):
            out[i][j] = iA[i][j]
        for j in range(n - h):
            out[i][h + j] = -X[i][j]
    for i in range(n - h):
        for j in range(n - h):
            out[h + i][h + j] = iB[i][j]
    return out


def _lu_solve_vec(g, rhs, n):
    a = [[(g[i][j] if j <= i else g[j][i]) for j in range(n)]
         for i in range(n)]
    b = [rhs[i] for i in range(n)]
    for k in range(n):
        mx = jnp.abs(a[k][k])
        for i in range(k + 1, n):
            mx = jnp.maximum(mx, jnp.abs(a[i][k]))
        taken = jnp.zeros_like(mx, dtype=bool)
        newk = [jnp.zeros_like(a[k][j]) for j in range(n)]
        newbk = jnp.zeros_like(b[k])
        for i in range(k, n):
            is_p = (jnp.abs(a[i][k]) == mx) & (~taken)
            taken = taken | is_p
            for j in range(n):
                newk[j] = jnp.where(is_p, a[i][j], newk[j])
            newbk = jnp.where(is_p, b[i], newbk)
            for j in range(n):
                a[i][j] = jnp.where(is_p, a[k][j], a[i][j])
            b[i] = jnp.where(is_p, b[k], b[i])
        for j in range(n):
            a[k][j] = newk[j]
        b[k] = newbk
        inv = 1.0 / a[k][k]
        for i in range(k + 1, n):
            mfac = a[i][k] * inv
            a[i][k] = mfac
            for j in range(k + 1, n):
                a[i][j] = a[i][j] - mfac * a[k][j]
    one = jnp.ones_like(a[0][0])
    zero = jnp.zeros_like(a[0][0])
    Lmat = [[a[i][j] if j < i else (one if j == i else zero)
             for j in range(n)] for i in range(n)]
    Umat = [[a[i][j] if j >= i else zero for j in range(n)]
            for i in range(n)]
    iL = _inv_unit_lower(Lmat, n)
    zv = [_matmul_sc(iL, [[v] for v in b], n, 1, n)[i][0] for i in range(n)]
    iU = _inv_upper(Umat, n)
    return [_matmul_sc(iU, [[v] for v in zv], n, 1, n)[i][0]
            for i in range(n)]


def _omp_block(kt, Dm, Dt):
    Bb = kt.shape[0]
    f32 = jnp.float32
    bf16 = jnp.bfloat16
    iota = lax.broadcasted_iota(jnp.int32, (Bb, N), 1)
    D16 = Dm.astype(bf16)

    r = kt
    idxs = []
    dcols = []
    onehots = []
    rhs = []
    g = [[None] * S for _ in range(S)]
    alpha0 = None
    x = None
    recon = None
    for t in range(S):
        proj = jnp.dot(r.astype(bf16), D16,
                       preferred_element_type=f32)
        if t == 0:
            alpha0 = proj
        a = jnp.abs(proj)
        m = jnp.max(a, axis=1, keepdims=True)
        idx = jnp.min(jnp.where(a == m, iota, N), axis=1)
        idxs.append(idx)
        ohb = iota == idx[:, None]
        onehot = ohb.astype(f32)
        dcol = jnp.dot(onehot, Dt, preferred_element_type=f32,
                       precision=lax.Precision.HIGHEST)
        dcols.append(dcol)
        onehots.append(ohb)
        rhs.append(jnp.sum(jnp.where(ohb, alpha0, 0.0), axis=1))
        grow = jnp.dot(dcol.astype(bf16), D16, preferred_element_type=f32)
        for j in range(t + 1):
            gv = jnp.sum(jnp.where(onehots[j], grow, 0.0), axis=1)
            if j == t:
                gv = gv + 1e-6
            g[t][j] = gv
        x = _lu_solve_vec(g, rhs, t + 1)
        recon = dcols[0] * x[0][:, None]
        for j in range(1, t + 1):
            recon = recon + dcols[j] * x[j][:, None]
        r = kt - recon
    return onehots, x, recon, r


def _kernel_body(kt_ref, d_ref, dt_ref, y_ref, khat_ref, loss_ref):
    kt = kt_ref[0]
    Dm = d_ref[0]
    masks, x, recon, resid = _omp_block(kt, Dm, dt_ref[0])

    y = jnp.zeros((kt.shape[0], N), jnp.float32)
    for t in range(S):
        y = jnp.where(masks[t], x[t][:, None], y)
    y_ref[...] = y
    khat_ref[0] = recon

    li = pl.program_id(0)
    bi = pl.program_id(1)

    @pl.when((li == 0) & (bi == 0))
    def _():
        loss_ref[0, 0] = 0.0

    loss_ref[0, 0] += jnp.sum(resid * resid)

    @pl.when((li == L - 1) & (bi == pl.num_programs(1) - 1))
    def _():
        loss_ref[0, 0] = loss_ref[0, 0] / (B * L * M)


@functools.partial(jax.jit, static_argnames=("interpret",))
def kernel(k, D, interpret=False):
    ktr = jnp.transpose(k, (1, 0, 2))
    Dt = jnp.transpose(D, (0, 2, 1))
    nb = B // BB
    y, khat, loss = pl.pallas_call(
        _kernel_body,
        grid=(L, nb),
        in_specs=[
            pl.BlockSpec((1, BB, M), lambda l, b: (l, b, 0)),
            pl.BlockSpec((1, M, N), lambda l, b: (l, 0, 0)),
            pl.BlockSpec((1, N, M), lambda l, b: (l, 0, 0)),
        ],
        out_specs=[
            pl.BlockSpec((BB, N), lambda l, b: (b, l)),
            pl.BlockSpec((1, BB, M), lambda l, b: (l, b, 0)),
            pl.BlockSpec(memory_space=pltpu.SMEM, block_shape=(1, 1),
                         index_map=lambda l, b: (0, 0)),
        ],
        out_shape=[
            jax.ShapeDtypeStruct((B, L * N), jnp.float32),
            jax.ShapeDtypeStruct((L, B, M), jnp.float32),
            jax.ShapeDtypeStruct((1, 1), jnp.float32),
        ],
        interpret=interpret,
    )(ktr, D, Dt)
    return (loss[0, 0], jnp.transpose(khat, (1, 0, 2)),
            jnp.reshape(y, (B, L, N)))
